# baseline (speedup 1.0000x reference)
import functools

import jax
import jax.numpy as jnp
import numpy as np
from jax import lax
from jax.experimental import pallas as pl
from jax.experimental.pallas import tpu as pltpu
from jax.experimental.pallas import tpu_sc as plsc

N = 10000
E = 320000
D = 128
H = 128
HW = H // 2
ET = 4

NC = 2
NS = 16
NW = NC * NS
EW = E // NW
KS = 40
NCHS = EW // KS
KR = 80
NCHR = EW // KR
CHR = 40
NRCH = N // CHR
LANES = 16
NG = H // 32

BN = 1000
BE = 4000

_IDX = np.arange(H).reshape(NG, 32)
_PACKPERM = np.concatenate([_IDX[:, :LANES].ravel(), _IDX[:, LANES:].ravel()])

QC = 8192.0


def _pack_rows(full, q):
    iq = jnp.clip(jnp.round(full * q), -32767.0, 32767.0).astype(jnp.int32)
    lo = iq[:, :HW]
    hi = iq[:, HW:]
    return (hi << 16) | (lo & jnp.int32(0xFFFF))


def _dense0_body(x_ref, ex_ref, wc_ref, bc_ref, wa_ref, wb_ref,
                 a_ref, b_ref, s_ref):
    cf = jnp.maximum(
        jnp.dot(x_ref[...], wc_ref[...], preferred_element_type=jnp.float32)
        + bc_ref[...], 0.0)
    cf = cf * ex_ref[...]
    a_ref[...] = jnp.dot(cf, wa_ref[...], preferred_element_type=jnp.float32)
    b_ref[...] = jnp.dot(cf, wb_ref[...], preferred_element_type=jnp.float32)

    @pl.when(pl.program_id(0) == 0)
    def _():
        s_ref[...] = jnp.zeros_like(s_ref)
    s_ref[...] += jnp.sum(cf, axis=0, keepdims=True)


def _dense0(x, ex, wc, bc, wa, wb):
    return pl.pallas_call(
        _dense0_body,
        grid=(N // BN,),
        in_specs=[
            pl.BlockSpec((BN, D), lambda i: (i, 0)),
            pl.BlockSpec((BN, 1), lambda i: (i, 0)),
            pl.BlockSpec((D, H), lambda i: (0, 0)),
            pl.BlockSpec((1, H), lambda i: (0, 0)),
            pl.BlockSpec((H, H), lambda i: (0, 0)),
            pl.BlockSpec((H, H), lambda i: (0, 0)),
        ],
        out_specs=[
            pl.BlockSpec((BN, H), lambda i: (i, 0)),
            pl.BlockSpec((BN, H), lambda i: (i, 0)),
            pl.BlockSpec((1, H), lambda i: (0, 0)),
        ],
        out_shape=[
            jax.ShapeDtypeStruct((N, H), jnp.float32),
            jax.ShapeDtypeStruct((N, H), jnp.float32),
            jax.ShapeDtypeStruct((1, H), jnp.float32),
        ],
    )(x, ex, wc, bc, wa, wb)


def _edgeC_body(ef_ref, w_ref, b_ref, c_ref):
    c_ref[...] = (jnp.dot(ef_ref[...], w_ref[...],
                          preferred_element_type=jnp.float32) + b_ref[...])


def _edgeC(ef, w, b):
    return pl.pallas_call(
        _edgeC_body,
        grid=(E // BE,),
        in_specs=[
            pl.BlockSpec((BE, ET), lambda i: (i, 0)),
            pl.BlockSpec((ET, H), lambda i: (0, 0)),
            pl.BlockSpec((1, H), lambda i: (0, 0)),
        ],
        out_specs=[
            pl.BlockSpec((BE, H), lambda i: (i, 0)),
        ],
        out_shape=[
            jax.ShapeDtypeStruct((E, H), jnp.float32),
        ],
    )(ef, w, b)


def _relu_sum_group(ra, rb, rc, r, g):
    s0 = pl.ds(g * 32, LANES)
    s1 = pl.ds(g * 32 + LANES, LANES)
    e0 = jnp.maximum(ra[r, s0] + rb[r, s0] + rc[r, s0], 0.0)
    e1 = jnp.maximum(ra[r, s1] + rb[r, s1] + rc[r, s1], 0.0)
    return e0, e1, s0, s1


def _stage_idx(spk, sbuf, dbuf, t, k, offsets):
    for o in offsets:
        v = spk[pl.ds(t * k + o, LANES)]
        sbuf[pl.ds(o, LANES)] = v & jnp.int32(0xFFFF)
        dbuf[pl.ds(o, LANES)] = v >> 16


@functools.cache
def _sc_scatter_kernel():
    return pl.kernel(
        _sc_scatter_body,
        out_type=jax.ShapeDtypeStruct((NC, N, H), jnp.float32),
        mesh=plsc.VectorSubcoreMesh(core_axis_name="c", subcore_axis_name="s"),
        scratch_types=[
            pltpu.VMEM_SHARED((N, H), jnp.float32),
            pltpu.VMEM((EW,), jnp.int32),
            pltpu.VMEM((KS,), jnp.int32),
            pltpu.VMEM((KS,), jnp.int32),
            pltpu.VMEM((KS,), jnp.int32),
            pltpu.VMEM((KS,), jnp.int32),
            pltpu.VMEM((KS, H), jnp.float32),
            pltpu.VMEM((KS, H), jnp.float32),
            pltpu.VMEM((KS, H), jnp.float32),
            pltpu.VMEM((KS, H), jnp.float32),
            pltpu.VMEM((KS, H), jnp.float32),
            pltpu.VMEM((KS, H), jnp.float32),
            pltpu.SemaphoreType.DMA,
            pltpu.SemaphoreType.DMA,
            pltpu.SemaphoreType.DMA,
            pltpu.SemaphoreType.DMA,
            pltpu.SemaphoreType.DMA,
            pltpu.SemaphoreType.DMA,
        ],
    )


def _sc_scatter(a0, b0, c0, spack):
    return _sc_scatter_kernel()(a0, b0, c0, spack)


def _sc_scatter_body(a_hbm, b_hbm, c_hbm, spk_hbm, out_hbm,
                     acc_sh, spk, si0, di0, si1, di1,
                     ra0, rb0, rc0, ra1, rb1, rc1,
                     sa0, sb0, sc0, sa1, sb1, sc1):
    c = lax.axis_index("c")
    s = lax.axis_index("s")
    wid = c * NS + s
    base0 = wid * EW
    bufs = ((ra0, rb0, rc0, si0, di0, sa0, sb0, sc0),
            (ra1, rb1, rc1, si1, di1, sa1, sb1, sc1))

    pltpu.sync_copy(spk_hbm.at[wid], spk)

    def zrow(i, _):
        for j in range(H // LANES):
            ra0[i, pl.ds(j * LANES, LANES)] = jnp.zeros((LANES,), jnp.float32)
        return 0
    lax.fori_loop(0, CHR, zrow, 0)

    def zcp(k, _):
        cid = s + k * NS

        @pl.when(cid < NRCH)
        def _():
            pltpu.sync_copy(ra0, acc_sh.at[pl.ds(cid * CHR, CHR)])
        return 0
    lax.fori_loop(0, pl.cdiv(NRCH, NS), zcp, 0)
    plsc.subcore_barrier()

    def issue(t, bi):
        ra, rb, rc, si, di, sa, sb, sc_ = bufs[bi]
        _stage_idx(spk, si, di, t, KS, (0, LANES, KS - LANES))
        pltpu.async_copy(a_hbm.at[si], ra, sa)
        pltpu.async_copy(b_hbm.at[di], rb, sb)
        pltpu.async_copy(c_hbm.at[pl.ds(base0 + t * KS, KS)], rc, sc_)

    def process(t, bi):
        ra, rb, rc, si, di, sa, sb, sc_ = bufs[bi]
        pltpu.make_async_copy(a_hbm.at[si], ra, sa).wait()
        pltpu.make_async_copy(b_hbm.at[di], rb, sb).wait()
        pltpu.make_async_copy(c_hbm.at[pl.ds(base0 + t * KS, KS)], rc,
                              sc_).wait()

        def erow(i, _):
            for u in range(2):
                r = 2 * i + u
                for g in range(NG):
                    e0, e1, s0, s1 = _relu_sum_group(ra, rb, rc, r, g)
                    ra[r, s0] = e0
                    ra[r, s1] = e1
            return 0
        lax.fori_loop(0, KS // 2, erow, 0)

        pltpu.sync_copy(ra, acc_sh.at[si], add=True)

    issue(0, 0)
    issue(1, 1)

    def pair(t2, _):
        t0 = 2 * t2
        process(t0, 0)

        @pl.when(t0 + 2 < NCHS)
        def _():
            issue(t0 + 2, 0)
        process(t0 + 1, 1)

        @pl.when(t0 + 3 < NCHS)
        def _():
            issue(t0 + 3, 1)
        return 0
    lax.fori_loop(0, NCHS // 2, pair, 0)
    plsc.subcore_barrier()

    def ocp(k, _):
        cid = s + k * NS

        @pl.when(cid < NRCH)
        def _():
            pltpu.sync_copy(acc_sh.at[pl.ds(cid * CHR, CHR)],
                            out_hbm.at[c, pl.ds(cid * CHR, CHR)])
        return 0
    lax.fori_loop(0, pl.cdiv(NRCH, NS), ocp, 0)


def _mid_body(p_ref, wa_ref, wb_ref, a_ref, b_ref, s_ref):
    cf = p_ref[0] + p_ref[1]
    a_ref[...] = jnp.dot(cf, wa_ref[...], preferred_element_type=jnp.float32)
    b_ref[...] = jnp.dot(cf, wb_ref[...], preferred_element_type=jnp.float32)

    @pl.when(pl.program_id(0) == 0)
    def _():
        s_ref[...] = jnp.zeros_like(s_ref)
    s_ref[...] += jnp.sum(cf, axis=0, keepdims=True)


def _mid(p, wa, wb):
    return pl.pallas_call(
        _mid_body,
        grid=(N // BN,),
        in_specs=[
            pl.BlockSpec((NC, BN, H), lambda i: (0, i, 0)),
            pl.BlockSpec((H, H), lambda i: (0, 0)),
            pl.BlockSpec((H, H), lambda i: (0, 0)),
        ],
        out_specs=[
            pl.BlockSpec((BN, H), lambda i: (i, 0)),
            pl.BlockSpec((BN, H), lambda i: (i, 0)),
            pl.BlockSpec((1, H), lambda i: (0, 0)),
        ],
        out_shape=[
            jax.ShapeDtypeStruct((N, H), jnp.float32),
            jax.ShapeDtypeStruct((N, H), jnp.float32),
            jax.ShapeDtypeStruct((1, H), jnp.float32),
        ],
    )(p, wa, wb)


@functools.cache
def _sc_reduce_kernel():
    return pl.kernel(
        _sc_reduce_body,
        out_type=jax.ShapeDtypeStruct((NW, H), jnp.float32),
        mesh=plsc.VectorSubcoreMesh(core_axis_name="c", subcore_axis_name="s"),
        scratch_types=[
            pltpu.VMEM((EW,), jnp.int32),
            pltpu.VMEM((KR,), jnp.int32),
            pltpu.VMEM((KR,), jnp.int32),
            pltpu.VMEM((KR,), jnp.int32),
            pltpu.VMEM((KR,), jnp.int32),
            pltpu.VMEM((KR, H), jnp.float32),
            pltpu.VMEM((KR, H), jnp.float32),
            pltpu.VMEM((KR, H), jnp.float32),
            pltpu.VMEM((KR, H), jnp.float32),
            pltpu.VMEM((KR, H), jnp.float32),
            pltpu.VMEM((KR, H), jnp.float32),
            pltpu.VMEM((H,), jnp.float32),
            pltpu.SemaphoreType.DMA,
            pltpu.SemaphoreType.DMA,
            pltpu.SemaphoreType.DMA,
            pltpu.SemaphoreType.DMA,
            pltpu.SemaphoreType.DMA,
            pltpu.SemaphoreType.DMA,
        ],
    )


def _sc_reduce(a1, b1, c1, spack):
    return _sc_reduce_kernel()(a1, b1, c1, spack)


def _sc_reduce_body(a_hbm, b_hbm, c_hbm, spk_hbm, out_hbm,
                    spk, si0, di0, si1, di1,
                    ra0, rb0, rc0, ra1, rb1, rc1, sbuf,
                    sa0, sb0, sc0, sa1, sb1, sc1):
    c = lax.axis_index("c")
    s = lax.axis_index("s")
    wid = c * NS + s
    base0 = wid * EW
    bufs = ((ra0, rb0, rc0, si0, di0, sa0, sb0, sc0),
            (ra1, rb1, rc1, si1, di1, sa1, sb1, sc1))

    pltpu.sync_copy(spk_hbm.at[wid], spk)

    def issue(t, bi):
        ra, rb, rc, si, di, sa, sb, sc_ = bufs[bi]
        _stage_idx(spk, si, di, t, KR, tuple(range(0, KR, LANES)))
        pltpu.async_copy(a_hbm.at[si], ra, sa)
        pltpu.async_copy(b_hbm.at[di], rb, sb)
        pltpu.async_copy(c_hbm.at[pl.ds(base0 + t * KR, KR)], rc, sc_)

    def process(t, bi, acc):
        ra, rb, rc, si, di, sa, sb, sc_ = bufs[bi]
        pltpu.make_async_copy(a_hbm.at[si], ra, sa).wait()
        pltpu.make_async_copy(b_hbm.at[di], rb, sb).wait()
        pltpu.make_async_copy(c_hbm.at[pl.ds(base0 + t * KR, KR)], rc,
                              sc_).wait()

        def erow(i, acc):
            new = list(acc)
            for u in range(2):
                r = 2 * i + u
                for g in range(NG):
                    e0, e1, _, _ = _relu_sum_group(ra, rb, rc, r, g)
                    new[2 * g] = new[2 * g] + e0
                    new[2 * g + 1] = new[2 * g + 1] + e1
            return tuple(new)
        return lax.fori_loop(0, KR // 2, erow, acc)

    issue(0, 0)
    issue(1, 1)
    acc0 = tuple(jnp.zeros((LANES,), jnp.float32) for _ in range(H // LANES))

    def pair(t2, acc):
        t0 = 2 * t2
        acc = process(t0, 0, acc)
        issue(t0 + 2, 0)
        acc = process(t0 + 1, 1, acc)

        @pl.when(t0 + 3 < NCHR)
        def _():
            issue(t0 + 3, 1)
        return acc
    acc = lax.fori_loop(0, (NCHR - 1) // 2, pair, acc0)
    acc = process(NCHR - 1, 0, acc)
    for j in range(H // LANES):
        sbuf[pl.ds(j * LANES, LANES)] = acc[j]
    pltpu.sync_copy(sbuf, out_hbm.at[wid])


def _post_body(s0_ref, s1_ref, s2p_ref, wp0_ref, wp1_ref, wp2_ref, bp_ref,
               o_ref):
    s2 = jnp.sum(s2p_ref[...], axis=0, keepdims=True)
    acc = (jnp.dot(s0_ref[...], wp0_ref[...], preferred_element_type=jnp.float32)
           + jnp.dot(s1_ref[...], wp1_ref[...], preferred_element_type=jnp.float32)
           + jnp.dot(s2, wp2_ref[...], preferred_element_type=jnp.float32)
           + bp_ref[...])
    o_ref[...] = jnp.maximum(acc, 0.0)


def _post(s0, s1, s2p, wp0, wp1, wp2, bp):
    return pl.pallas_call(
        _post_body,
        out_shape=jax.ShapeDtypeStruct((1, D), jnp.float32),
    )(s0, s1, s2p, wp0, wp1, wp2, bp)


def kernel(child_feats, child_exists, edge_type_onehot, edge_indices,
           W_child, b_child, W_ne0, b_ne0, W_ne1, b_ne1, W_parent, b_parent):
    x = child_feats[0]
    ex = child_exists[0]
    ef = edge_type_onehot[0]
    ei = edge_indices[0]
    spack = (ei[:, 0] | (ei[:, 1] << 16)).reshape(NW, EW)

    w0a, w0b, w0c = W_ne0[:H], W_ne0[H:2 * H], W_ne0[2 * H:]
    w1a, w1b, w1c = W_ne1[:H], W_ne1[H:2 * H], W_ne1[2 * H:]
    wp0, wp1, wp2 = W_parent[:H], W_parent[H:2 * H], W_parent[2 * H:]
    bc = b_child[None]
    b0 = b_ne0[None]
    b1 = b_ne1[None]
    bp = b_parent[None]

    a0, b0rows, s0 = _dense0(x, ex, W_child, bc, w0a, w0b)
    (c0,) = _edgeC(ef, w0c, b0)
    p = _sc_scatter(a0, b0rows, c0, spack)
    (c1,) = _edgeC(ef, w1c, b1)
    a1, b1rows, s1 = _mid(p, w1a, w1b)
    s2p = _sc_reduce(a1, b1rows, c1, spack)
    return _post(s0, s1, s2p, wp0, wp1, wp2, bp)


# speedup vs baseline: 1.0464x; 1.0464x over previous
import functools

import jax
import jax.numpy as jnp
import numpy as np
from jax import lax
from jax.experimental import pallas as pl
from jax.experimental.pallas import tpu as pltpu
from jax.experimental.pallas import tpu_sc as plsc

N = 10000
E = 320000
D = 128
H = 128
HW = H // 2
ET = 4

NC = 2
NS = 16
NW = NC * NS
EW = E // NW
KS = 40
NCHS = EW // KS
KR = 80
NCHR = EW // KR
CHR = 40
NRCH = N // CHR
LANES = 16
NG = H // 32

BN = 1000
BE = 4000

_IDX = np.arange(H).reshape(NG, 32)
_PACKPERM = np.concatenate([_IDX[:, :LANES].ravel(), _IDX[:, LANES:].ravel()])

QC = 8192.0


def _pack_rows(full, q):
    iq = jnp.clip(jnp.round(full * q), -32767.0, 32767.0).astype(jnp.int32)
    lo = iq[:, :HW]
    hi = iq[:, HW:]
    return (hi << 16) | (lo & jnp.int32(0xFFFF))


def _dense0_body(x_ref, ex_ref, wc_ref, bc_ref, wa_ref, wb_ref,
                 a_ref, b_ref, s_ref):
    cf = jnp.maximum(
        jnp.dot(x_ref[...], wc_ref[...], preferred_element_type=jnp.float32)
        + bc_ref[...], 0.0)
    cf = cf * ex_ref[...]
    a_ref[...] = jnp.dot(cf, wa_ref[...], preferred_element_type=jnp.float32)
    b_ref[...] = jnp.dot(cf, wb_ref[...], preferred_element_type=jnp.float32)

    @pl.when(pl.program_id(0) == 0)
    def _():
        s_ref[...] = jnp.zeros_like(s_ref)
    s_ref[...] += jnp.sum(cf, axis=0, keepdims=True)


def _dense0(x, ex, wc, bc, wa, wb):
    return pl.pallas_call(
        _dense0_body,
        grid=(N // BN,),
        in_specs=[
            pl.BlockSpec((BN, D), lambda i: (i, 0)),
            pl.BlockSpec((BN, 1), lambda i: (i, 0)),
            pl.BlockSpec((D, H), lambda i: (0, 0)),
            pl.BlockSpec((1, H), lambda i: (0, 0)),
            pl.BlockSpec((H, H), lambda i: (0, 0)),
            pl.BlockSpec((H, H), lambda i: (0, 0)),
        ],
        out_specs=[
            pl.BlockSpec((BN, H), lambda i: (i, 0)),
            pl.BlockSpec((BN, H), lambda i: (i, 0)),
            pl.BlockSpec((1, H), lambda i: (0, 0)),
        ],
        out_shape=[
            jax.ShapeDtypeStruct((N, H), jnp.float32),
            jax.ShapeDtypeStruct((N, H), jnp.float32),
            jax.ShapeDtypeStruct((1, H), jnp.float32),
        ],
    )(x, ex, wc, bc, wa, wb)


def _edgeC_body(ef_ref, w0_ref, b0_ref, w1_ref, b1_ref, c0_ref, c1_ref):
    ef = ef_ref[...]
    c0_ref[...] = (jnp.dot(ef, w0_ref[...], preferred_element_type=jnp.float32)
                   + b0_ref[...])
    c1_ref[...] = (jnp.dot(ef, w1_ref[...], preferred_element_type=jnp.float32)
                   + b1_ref[...])


def _edgeC(ef, w0, b0, w1, b1):
    return pl.pallas_call(
        _edgeC_body,
        grid=(E // BE,),
        in_specs=[
            pl.BlockSpec((BE, ET), lambda i: (i, 0)),
            pl.BlockSpec((ET, H), lambda i: (0, 0)),
            pl.BlockSpec((1, H), lambda i: (0, 0)),
            pl.BlockSpec((ET, H), lambda i: (0, 0)),
            pl.BlockSpec((1, H), lambda i: (0, 0)),
        ],
        out_specs=[
            pl.BlockSpec((BE, H), lambda i: (i, 0)),
            pl.BlockSpec((BE, H), lambda i: (i, 0)),
        ],
        out_shape=[
            jax.ShapeDtypeStruct((E, H), jnp.float32),
            jax.ShapeDtypeStruct((E, H), jnp.float32),
        ],
    )(ef, w0, b0, w1, b1)


def _relu_sum_group(ra, rb, rc, r, g):
    s0 = pl.ds(g * 32, LANES)
    s1 = pl.ds(g * 32 + LANES, LANES)
    e0 = jnp.maximum(ra[r, s0] + rb[r, s0] + rc[r, s0], 0.0)
    e1 = jnp.maximum(ra[r, s1] + rb[r, s1] + rc[r, s1], 0.0)
    return e0, e1, s0, s1


def _stage_idx(spk, sbuf, dbuf, base, offsets):
    for o in offsets:
        v = spk[pl.ds(base + o, LANES)]
        sbuf[pl.ds(o, LANES)] = v & jnp.int32(0xFFFF)
        dbuf[pl.ds(o, LANES)] = v >> 16


@functools.cache
def _sc_scatter_kernel():
    return pl.kernel(
        _sc_scatter_body,
        out_type=jax.ShapeDtypeStruct((NC, N, H), jnp.float32),
        mesh=plsc.VectorSubcoreMesh(core_axis_name="c", subcore_axis_name="s"),
        scratch_types=[
            pltpu.VMEM_SHARED((N, H), jnp.float32),
            pltpu.VMEM((5008,), jnp.int32),
            pltpu.VMEM((KS,), jnp.int32),
            pltpu.VMEM((KS,), jnp.int32),
            pltpu.VMEM((KS,), jnp.int32),
            pltpu.VMEM((KS,), jnp.int32),
            pltpu.VMEM((KS, H), jnp.float32),
            pltpu.VMEM((KS, H), jnp.float32),
            pltpu.VMEM((KS, H), jnp.float32),
            pltpu.VMEM((KS, H), jnp.float32),
            pltpu.VMEM((KS, H), jnp.float32),
            pltpu.VMEM((KS, H), jnp.float32),
            pltpu.VMEM((KS, H), jnp.float32),
            pltpu.VMEM((KS, H), jnp.float32),
            pltpu.SemaphoreType.DMA,
            pltpu.SemaphoreType.DMA,
            pltpu.SemaphoreType.DMA,
            pltpu.SemaphoreType.DMA,
            pltpu.SemaphoreType.DMA,
            pltpu.SemaphoreType.DMA,
            pltpu.SemaphoreType.DMA,
            pltpu.SemaphoreType.DMA,
        ],
    )


def _sc_scatter(a0, b0, c0, spack):
    return _sc_scatter_kernel()(a0, b0, c0, spack)


def _sc_scatter_body(a_hbm, b_hbm, c_hbm, spk_hbm, out_hbm,
                     acc_sh, spk, si0, di0, si1, di1,
                     ra0, rb0, rc0, ra1, rb1, rc1, nef0, nef1,
                     sa0, sb0, sc0, sa1, sb1, sc1, sw0, sw1):
    c = lax.axis_index("c")
    s = lax.axis_index("s")
    wid = c * NS + s
    base0 = wid * EW
    bufs = ((ra0, rb0, rc0, si0, di0, nef0, sa0, sb0, sc0, sw0),
            (ra1, rb1, rc1, si1, di1, nef1, sa1, sb1, sc1, sw1))

    pltpu.sync_copy(spk_hbm.at[pl.ds(wid * EW, 5008)], spk)

    def zrow(i, _):
        for j in range(H // LANES):
            ra0[i, pl.ds(j * LANES, LANES)] = jnp.zeros((LANES,), jnp.float32)
        return 0
    lax.fori_loop(0, CHR, zrow, 0)

    def zcp(k, _):
        cid = s + k * NS

        @pl.when(cid < NRCH)
        def _():
            pltpu.sync_copy(ra0, acc_sh.at[pl.ds(cid * CHR, CHR)])
        return 0
    lax.fori_loop(0, pl.cdiv(NRCH, NS), zcp, 0)
    plsc.subcore_barrier()

    def issue(t, bi):
        ra, rb, rc, si, di, nef, sa, sb, sc_, sw = bufs[bi]

        @pl.when(t == NCHS // 2)
        def _():
            pltpu.sync_copy(spk_hbm.at[pl.ds(wid * EW + 4992, 5008)], spk)
        base = jnp.where(t >= NCHS // 2, t * KS - 4992, t * KS)
        _stage_idx(spk, si, di, base, (0, LANES, KS - LANES))
        pltpu.async_copy(a_hbm.at[si], ra, sa)
        pltpu.async_copy(b_hbm.at[di], rb, sb)
        pltpu.async_copy(c_hbm.at[pl.ds(base0 + t * KS, KS)], rc, sc_)

    def process(t, bi, first):
        ra, rb, rc, si, di, nef, sa, sb, sc_, sw = bufs[bi]
        pltpu.make_async_copy(a_hbm.at[si], ra, sa).wait()
        pltpu.make_async_copy(b_hbm.at[di], rb, sb).wait()
        pltpu.make_async_copy(c_hbm.at[pl.ds(base0 + t * KS, KS)], rc,
                              sc_).wait()

        def erow(i, _):
            for u in range(2):
                r = 2 * i + u
                for g in range(NG):
                    e0, e1, s0, s1 = _relu_sum_group(ra, rb, rc, r, g)
                    nef[r, s0] = e0
                    nef[r, s1] = e1
            return 0
        lax.fori_loop(0, KS // 2, erow, 0)

        @pl.when(jnp.logical_not(first))
        def _():
            pltpu.make_async_copy(nef, acc_sh.at[si], sw).wait()
        pltpu.async_copy(nef, acc_sh.at[si], sw, add=True)

    issue(0, 0)
    issue(1, 1)

    def pair(t2, _):
        t0 = 2 * t2
        process(t0, 0, t2 == 0)

        @pl.when(t0 + 2 < NCHS)
        def _():
            issue(t0 + 2, 0)
        process(t0 + 1, 1, t2 == 0)

        @pl.when(t0 + 3 < NCHS)
        def _():
            issue(t0 + 3, 1)
        return 0
    lax.fori_loop(0, NCHS // 2, pair, 0)
    pltpu.make_async_copy(nef0, acc_sh.at[si0], sw0).wait()
    pltpu.make_async_copy(nef1, acc_sh.at[si1], sw1).wait()
    plsc.subcore_barrier()

    def ocp(k, _):
        cid = s + k * NS

        @pl.when(cid < NRCH)
        def _():
            pltpu.sync_copy(acc_sh.at[pl.ds(cid * CHR, CHR)],
                            out_hbm.at[c, pl.ds(cid * CHR, CHR)])
        return 0
    lax.fori_loop(0, pl.cdiv(NRCH, NS), ocp, 0)


def _mid_body(p_ref, wa_ref, wb_ref, a_ref, b_ref, s_ref):
    cf = p_ref[0] + p_ref[1]
    a_ref[...] = jnp.dot(cf, wa_ref[...], preferred_element_type=jnp.float32)
    b_ref[...] = jnp.dot(cf, wb_ref[...], preferred_element_type=jnp.float32)

    @pl.when(pl.program_id(0) == 0)
    def _():
        s_ref[...] = jnp.zeros_like(s_ref)
    s_ref[...] += jnp.sum(cf, axis=0, keepdims=True)


def _mid(p, wa, wb):
    return pl.pallas_call(
        _mid_body,
        grid=(N // BN,),
        in_specs=[
            pl.BlockSpec((NC, BN, H), lambda i: (0, i, 0)),
            pl.BlockSpec((H, H), lambda i: (0, 0)),
            pl.BlockSpec((H, H), lambda i: (0, 0)),
        ],
        out_specs=[
            pl.BlockSpec((BN, H), lambda i: (i, 0)),
            pl.BlockSpec((BN, H), lambda i: (i, 0)),
            pl.BlockSpec((1, H), lambda i: (0, 0)),
        ],
        out_shape=[
            jax.ShapeDtypeStruct((N, H), jnp.float32),
            jax.ShapeDtypeStruct((N, H), jnp.float32),
            jax.ShapeDtypeStruct((1, H), jnp.float32),
        ],
    )(p, wa, wb)


@functools.cache
def _sc_reduce_kernel():
    return pl.kernel(
        _sc_reduce_body,
        out_type=jax.ShapeDtypeStruct((NW, H), jnp.float32),
        mesh=plsc.VectorSubcoreMesh(core_axis_name="c", subcore_axis_name="s"),
        scratch_types=[
            pltpu.VMEM((EW,), jnp.int32),
            pltpu.VMEM((KR,), jnp.int32),
            pltpu.VMEM((KR,), jnp.int32),
            pltpu.VMEM((KR,), jnp.int32),
            pltpu.VMEM((KR,), jnp.int32),
            pltpu.VMEM((KR, H), jnp.float32),
            pltpu.VMEM((KR, H), jnp.float32),
            pltpu.VMEM((KR, H), jnp.float32),
            pltpu.VMEM((KR, H), jnp.float32),
            pltpu.VMEM((KR, H), jnp.float32),
            pltpu.VMEM((KR, H), jnp.float32),
            pltpu.VMEM((H,), jnp.float32),
            pltpu.SemaphoreType.DMA,
            pltpu.SemaphoreType.DMA,
            pltpu.SemaphoreType.DMA,
            pltpu.SemaphoreType.DMA,
            pltpu.SemaphoreType.DMA,
            pltpu.SemaphoreType.DMA,
        ],
    )


def _sc_reduce(a1, b1, c1, spack):
    return _sc_reduce_kernel()(a1, b1, c1, spack)


def _sc_reduce_body(a_hbm, b_hbm, c_hbm, spk_hbm, out_hbm,
                    spk, si0, di0, si1, di1,
                    ra0, rb0, rc0, ra1, rb1, rc1, sbuf,
                    sa0, sb0, sc0, sa1, sb1, sc1):
    c = lax.axis_index("c")
    s = lax.axis_index("s")
    wid = c * NS + s
    base0 = wid * EW
    bufs = ((ra0, rb0, rc0, si0, di0, sa0, sb0, sc0),
            (ra1, rb1, rc1, si1, di1, sa1, sb1, sc1))

    pltpu.sync_copy(spk_hbm.at[pl.ds(wid * EW, EW)], spk)

    def issue(t, bi):
        ra, rb, rc, si, di, sa, sb, sc_ = bufs[bi]
        _stage_idx(spk, si, di, t * KR, tuple(range(0, KR, LANES)))
        pltpu.async_copy(a_hbm.at[si], ra, sa)
        pltpu.async_copy(b_hbm.at[di], rb, sb)
        pltpu.async_copy(c_hbm.at[pl.ds(base0 + t * KR, KR)], rc, sc_)

    def process(t, bi, acc):
        ra, rb, rc, si, di, sa, sb, sc_ = bufs[bi]
        pltpu.make_async_copy(a_hbm.at[si], ra, sa).wait()
        pltpu.make_async_copy(b_hbm.at[di], rb, sb).wait()
        pltpu.make_async_copy(c_hbm.at[pl.ds(base0 + t * KR, KR)], rc,
                              sc_).wait()

        def erow(i, acc):
            new = list(acc)
            for u in range(2):
                r = 2 * i + u
                for g in range(NG):
                    e0, e1, _, _ = _relu_sum_group(ra, rb, rc, r, g)
                    new[2 * g] = new[2 * g] + e0
                    new[2 * g + 1] = new[2 * g + 1] + e1
            return tuple(new)
        return lax.fori_loop(0, KR // 2, erow, acc)

    issue(0, 0)
    issue(1, 1)
    acc0 = tuple(jnp.zeros((LANES,), jnp.float32) for _ in range(H // LANES))

    def pair(t2, acc):
        t0 = 2 * t2
        acc = process(t0, 0, acc)
        issue(t0 + 2, 0)
        acc = process(t0 + 1, 1, acc)

        @pl.when(t0 + 3 < NCHR)
        def _():
            issue(t0 + 3, 1)
        return acc
    acc = lax.fori_loop(0, (NCHR - 1) // 2, pair, acc0)
    acc = process(NCHR - 1, 0, acc)
    for j in range(H // LANES):
        sbuf[pl.ds(j * LANES, LANES)] = acc[j]
    pltpu.sync_copy(sbuf, out_hbm.at[wid])


def _post_body(s0_ref, s1_ref, s2p_ref, wp0_ref, wp1_ref, wp2_ref, bp_ref,
               o_ref):
    s2 = jnp.sum(s2p_ref[...], axis=0, keepdims=True)
    acc = (jnp.dot(s0_ref[...], wp0_ref[...], preferred_element_type=jnp.float32)
           + jnp.dot(s1_ref[...], wp1_ref[...], preferred_element_type=jnp.float32)
           + jnp.dot(s2, wp2_ref[...], preferred_element_type=jnp.float32)
           + bp_ref[...])
    o_ref[...] = jnp.maximum(acc, 0.0)


def _post(s0, s1, s2p, wp0, wp1, wp2, bp):
    return pl.pallas_call(
        _post_body,
        out_shape=jax.ShapeDtypeStruct((1, D), jnp.float32),
    )(s0, s1, s2p, wp0, wp1, wp2, bp)


def kernel(child_feats, child_exists, edge_type_onehot, edge_indices,
           W_child, b_child, W_ne0, b_ne0, W_ne1, b_ne1, W_parent, b_parent):
    x = child_feats[0]
    ex = child_exists[0]
    ef = edge_type_onehot[0]
    ei = edge_indices[0]
    spack = ei[:, 0] | (ei[:, 1] << 16)

    w0a, w0b, w0c = W_ne0[:H], W_ne0[H:2 * H], W_ne0[2 * H:]
    w1a, w1b, w1c = W_ne1[:H], W_ne1[H:2 * H], W_ne1[2 * H:]
    wp0, wp1, wp2 = W_parent[:H], W_parent[H:2 * H], W_parent[2 * H:]
    bc = b_child[None]
    b0 = b_ne0[None]
    b1 = b_ne1[None]
    bp = b_parent[None]

    a0, b0rows, s0 = _dense0(x, ex, W_child, bc, w0a, w0b)
    c0, c1 = _edgeC(ef, w0c, b0, w1c, b1)
    p = _sc_scatter(a0, b0rows, c0, spack)
    a1, b1rows, s1 = _mid(p, w1a, w1b)
    s2p = _sc_reduce(a1, b1rows, c1, spack)
    return _post(s0, s1, s2p, wp0, wp1, wp2, bp)
